# baseline (speedup 1.0000x reference)
import functools
import math

import jax
import jax.numpy as jnp
from jax import lax
from jax.experimental import pallas as pl
from jax.experimental.pallas import tpu as pltpu
from jax.experimental.pallas import tpu_sc as plsc

NC = 2
NS = 16
NW = NC * NS
L = 16


@functools.lru_cache(maxsize=None)
def _build(T, B, V, D, C):
    n_units = T * (B // C)
    assert n_units % NW == 0 and D % L == 0 and C % L == 0
    u_per_w = n_units // NW
    nbb = B // C
    scale = float(math.sqrt(D))
    mesh = plsc.VectorSubcoreMesh(
        core_axis_name="c", subcore_axis_name="s",
        num_cores=NC, num_subcores=NS)

    @functools.partial(
        pl.kernel,
        out_type=jax.ShapeDtypeStruct((T, D, B), jnp.float32),
        mesh=mesh,
        scratch_types=[
            [pltpu.VMEM((C,), jnp.int32) for _ in range(2)],
            [pltpu.VMEM((C,), jnp.int32) for _ in range(2)],
            [pltpu.VMEM((C, 2 * D), jnp.float32) for _ in range(2)],
            [pltpu.VMEM((D, C), jnp.float32) for _ in range(2)],
            [pltpu.SemaphoreType.DMA for _ in range(2)],
            [pltpu.SemaphoreType.DMA for _ in range(2)],
            [pltpu.SemaphoreType.DMA for _ in range(2)],
        ],
        compiler_params=pltpu.CompilerParams(
            use_tc_tiling_on_sc=True, needs_layout_passes=False),
    )
    def emb(tokt_hbm, tab2_hbm, out_hbm, idx_bufs, pair_idx, pair_bufs,
            tr_bufs, isems, gsems, osems):
        wid = lax.axis_index("s") * NC + lax.axis_index("c")
        u_base = wid * u_per_w

        def unit_tb(u):
            return u // nbb, (u % nbb) * C

        def idx_copy(u, b):
            t, b0 = unit_tb(u)
            return pltpu.make_async_copy(
                tokt_hbm.at[t, pl.ds(b0, C)], idx_bufs[b], isems[b])

        def gather_copy(b):
            return pltpu.make_async_copy(
                tab2_hbm.at[pair_idx[b]], pair_bufs[b], gsems[b])

        def out_copy(u, b):
            t, b0 = unit_tb(u)
            return pltpu.make_async_copy(
                tr_bufs[b], out_hbm.at[t, :, pl.ds(b0, C)], osems[b])

        def prep_and_gather(b):
            @plsc.parallel_loop(0, C // L, 1, unroll=4)
            def _halve(k):
                sl = pl.ds(k * L, L)
                pair_idx[b][sl] = idx_bufs[b][sl] >> 1
            gather_copy(b).start()

        idx_copy(u_base, 0).start()
        idx_copy(u_base + 1, 1).start()
        idx_copy(u_base, 0).wait()
        prep_and_gather(0)

        assert u_per_w % 2 == 0

        def step(i2, carry):
            for b in range(2):
                i = i2 * 2 + b
                u = u_base + i
                nb = 1 - b
                @pl.when(i + 1 < u_per_w)
                def _feed_next():
                    idx_copy(u + 1, nb).wait()
                    prep_and_gather(nb)
                gather_copy(b).wait()

                @pl.when(i >= 2)
                def _drain_out():
                    out_copy(u - 2, b).wait()

                def col(c, carry):
                    @plsc.parallel_loop(0, C // L, 1, unroll=2)
                    def _grp(k):
                        sl = pl.ds(k * L, L)
                        rows = lax.iota(jnp.int32, L) + (k * L)
                        half = (idx_bufs[b][sl] & 1) * D + c
                        v = plsc.load_gather(pair_bufs[b], [rows, half])
                        tr_bufs[b][c, sl] = v * scale
                    return carry

                lax.fori_loop(0, D, col, 0)
                out_copy(u, b).start()
                @pl.when(i + 2 < u_per_w)
                def _stage_next():
                    idx_copy(u + 2, b).start()
            return carry

        lax.fori_loop(0, u_per_w // 2, step, 0)
        out_copy(u_base + u_per_w - 2, 0).wait()
        out_copy(u_base + u_per_w - 1, 1).wait()

    return emb


@jax.jit
def kernel(tokens, table):
    B, T = tokens.shape
    V, D = table.shape
    tokt = tokens.T.astype(jnp.int32)
    tab2 = table.reshape(V // 2, 2 * D)
    outt = _build(T, B, V, D, 256)(tokt, tab2)
    return outt.transpose(2, 0, 1)


# speedup vs baseline: 1.1032x; 1.1032x over previous
import functools
import math

import jax
import jax.numpy as jnp
from jax import lax
from jax.experimental import pallas as pl
from jax.experimental.pallas import tpu as pltpu
from jax.experimental.pallas import tpu_sc as plsc

NC = 2
NS = 16
NW = NC * NS
L = 16


@functools.lru_cache(maxsize=None)
def _build_row_table(V, D, W):
    scale = float(math.sqrt(D))
    grid = (pl.cdiv(V, W),)

    def body(t_ref, o_ref):
        x = t_ref[...]
        o_ref[:, 0:D] = jnp.swapaxes(x, 0, 1) * scale

    return pl.pallas_call(
        body,
        grid=grid,
        in_specs=[pl.BlockSpec((D, W), lambda i: (0, i))],
        out_specs=pl.BlockSpec((W, 2 * D), lambda i: (i, 0)),
        out_shape=jax.ShapeDtypeStruct((V, 2 * D), jnp.float32),
    )


@functools.lru_cache(maxsize=None)
def _build(T, B, V, D, C):
    n_units = T * (B // C)
    assert n_units % NW == 0 and D % L == 0 and C % L == 0
    u_per_w = n_units // NW
    nbb = B // C
    scale = float(math.sqrt(D))
    mesh = plsc.VectorSubcoreMesh(
        core_axis_name="c", subcore_axis_name="s",
        num_cores=NC, num_subcores=NS)

    @functools.partial(
        pl.kernel,
        out_type=jax.ShapeDtypeStruct((T, D, B), jnp.float32),
        mesh=mesh,
        scratch_types=[
            [pltpu.VMEM((C,), jnp.int32) for _ in range(2)],
            [pltpu.VMEM((C, 2 * D), jnp.float32) for _ in range(2)],
            [pltpu.VMEM((D, C), jnp.float32) for _ in range(2)],
            [pltpu.SemaphoreType.DMA for _ in range(2)],
            [pltpu.SemaphoreType.DMA for _ in range(2)],
            [pltpu.SemaphoreType.DMA for _ in range(2)],
        ],
        compiler_params=pltpu.CompilerParams(
            use_tc_tiling_on_sc=True, needs_layout_passes=False),
    )
    def emb(tokt_hbm, tab2_hbm, out_hbm, idx_bufs, pair_bufs,
            tr_bufs, isems, gsems, osems):
        wid = lax.axis_index("s") * NC + lax.axis_index("c")
        u_base = wid * u_per_w

        def unit_tb(u):
            return u // nbb, (u % nbb) * C

        def idx_copy(u, b):
            t, b0 = unit_tb(u)
            return pltpu.make_async_copy(
                tokt_hbm.at[t, pl.ds(b0, C)], idx_bufs[b], isems[b])

        def gather_copy(b):
            return pltpu.make_async_copy(
                tab2_hbm.at[idx_bufs[b]], pair_bufs[b], gsems[b])

        def out_copy(u, b):
            t, b0 = unit_tb(u)
            return pltpu.make_async_copy(
                tr_bufs[b], out_hbm.at[t, :, pl.ds(b0, C)], osems[b])

        idx_copy(u_base, 0).start()
        idx_copy(u_base + 1, 1).start()
        idx_copy(u_base, 0).wait()
        gather_copy(0).start()

        assert u_per_w % 2 == 0

        def step(i2, carry):
            for b in range(2):
                i = i2 * 2 + b
                u = u_base + i
                nb = 1 - b
                @pl.when(i + 1 < u_per_w)
                def _feed_next():
                    idx_copy(u + 1, nb).wait()
                    gather_copy(nb).start()
                gather_copy(b).wait()

                @pl.when(i >= 2)
                def _drain_out():
                    out_copy(u - 2, b).wait()

                def col(c, carry):
                    lanes = lax.iota(jnp.int32, L)
                    feat = (c + lanes) & (D - 1)

                    @plsc.parallel_loop(0, C // L, 1, unroll=2)
                    def _grp(k):
                        rows = lanes + (k * L)
                        v = plsc.load_gather(pair_bufs[b], [rows, feat])
                        plsc.store_scatter(tr_bufs[b], [feat, rows], v)
                    return carry

                lax.fori_loop(0, D, col, 0)
                out_copy(u, b).start()
                @pl.when(i + 2 < u_per_w)
                def _stage_next():
                    idx_copy(u + 2, b).start()
            return carry

        lax.fori_loop(0, u_per_w // 2, step, 0)
        out_copy(u_base + u_per_w - 2, 0).wait()
        out_copy(u_base + u_per_w - 1, 1).wait()

    return emb


@jax.jit
def kernel(tokens, table):
    B, T = tokens.shape
    V, D = table.shape
    tokt = tokens.T.astype(jnp.int32)
    tab2 = _build_row_table(V, D, 512)(table.T)
    outt = _build(T, B, V, D, 256)(tokt, tab2)
    return outt.transpose(2, 0, 1)


# speedup vs baseline: 2.2254x; 2.0172x over previous
import functools
import math

import jax
import jax.numpy as jnp
from jax import lax
from jax.experimental import pallas as pl
from jax.experimental.pallas import tpu as pltpu
from jax.experimental.pallas import tpu_sc as plsc

NC = 2
NS = 16
NW = NC * NS
L = 16


@functools.lru_cache(maxsize=None)
def _build(T, B, V, D, C):
    n_units = T * (B // C)
    assert n_units % NW == 0 and D % L == 0 and C % L == 0
    u_per_w = n_units // NW
    nbb = B // C
    scale = float(math.sqrt(D))
    mesh = plsc.VectorSubcoreMesh(
        core_axis_name="c", subcore_axis_name="s",
        num_cores=NC, num_subcores=NS)

    @functools.partial(
        pl.kernel,
        out_type=jax.ShapeDtypeStruct((T, D, B), jnp.float32),
        mesh=mesh,
        scratch_types=[
            [pltpu.VMEM((C,), jnp.int32) for _ in range(2)],
            [pltpu.VMEM((C, D), jnp.float32) for _ in range(2)],
            [pltpu.VMEM((D, C), jnp.float32) for _ in range(2)],
            [pltpu.SemaphoreType.DMA for _ in range(2)],
            [pltpu.SemaphoreType.DMA for _ in range(2)],
            [pltpu.SemaphoreType.DMA for _ in range(2)],
        ],
        compiler_params=pltpu.CompilerParams(
            use_tc_tiling_on_sc=True, needs_layout_passes=False),
    )
    def emb(tok_hbm, tab_hbm, out_hbm, idx_bufs, row_bufs,
            tr_bufs, isems, gsems, osems):
        wid = lax.axis_index("s") * NC + lax.axis_index("c")
        u_base = wid * u_per_w

        def unit_tb(u):
            return u // nbb, (u % nbb) * C

        def idx_copy(u, b):
            t, b0 = unit_tb(u)
            return pltpu.make_async_copy(
                tok_hbm.at[t, pl.ds(b0, C)], idx_bufs[b], isems[b])

        def start_row_fetches(b):
            def grp(k, carry):
                v16 = idx_bufs[b][pl.ds(k * L, L)]
                for j in range(L):
                    pltpu.make_async_copy(
                        tab_hbm.at[pl.ds(v16[j], 1), :],
                        row_bufs[b].at[pl.ds(k * L + j, 1), :],
                        gsems[b]).start()
                return carry
            lax.fori_loop(0, C // L, grp, 0)

        def drain_row_fetches(b):
            pltpu.make_async_copy(
                tab_hbm.at[pl.ds(0, C), :], row_bufs[b], gsems[b]).wait()

        def out_copy(u, b):
            t, b0 = unit_tb(u)
            return pltpu.make_async_copy(
                tr_bufs[b], out_hbm.at[t, :, pl.ds(b0, C)], osems[b])

        idx_copy(u_base, 0).start()
        idx_copy(u_base + 1, 1).start()
        idx_copy(u_base, 0).wait()
        start_row_fetches(0)

        assert u_per_w % 2 == 0

        def step(i2, carry):
            for b in range(2):
                i = i2 * 2 + b
                u = u_base + i
                nb = 1 - b
                @pl.when(i + 1 < u_per_w)
                def _feed_next():
                    idx_copy(u + 1, nb).wait()
                    start_row_fetches(nb)
                drain_row_fetches(b)

                @pl.when(i >= 2)
                def _drain_out():
                    out_copy(u - 2, b).wait()

                def col(c, carry):
                    lanes = lax.iota(jnp.int32, L)
                    feat = (c + lanes) & (D - 1)

                    @plsc.parallel_loop(0, C // L, 1, unroll=2)
                    def _grp(k):
                        rows = lanes + (k * L)
                        v = plsc.load_gather(row_bufs[b], [rows, feat])
                        plsc.store_scatter(tr_bufs[b], [feat, rows], v * scale)
                    return carry

                lax.fori_loop(0, D, col, 0)
                out_copy(u, b).start()
                @pl.when(i + 2 < u_per_w)
                def _stage_next():
                    idx_copy(u + 2, b).start()
            return carry

        lax.fori_loop(0, u_per_w // 2, step, 0)
        out_copy(u_base + u_per_w - 2, 0).wait()
        out_copy(u_base + u_per_w - 1, 1).wait()

    return emb


@jax.jit
def kernel(tokens, table):
    B, T = tokens.shape
    V, D = table.shape
    tokt = tokens.T.astype(jnp.int32)
    outt = _build(T, B, V, D, 256)(tokt, table)
    return outt.transpose(2, 0, 1)


# speedup vs baseline: 2.2414x; 1.0072x over previous
import functools
import math

import jax
import jax.numpy as jnp
from jax import lax
from jax.experimental import pallas as pl
from jax.experimental.pallas import tpu as pltpu
from jax.experimental.pallas import tpu_sc as plsc

NC = 2
NS = 16
NW = NC * NS
L = 16


@functools.lru_cache(maxsize=None)
def _build(T, B, V, D, C):
    n_units = T * (B // C)
    assert n_units % NW == 0 and D % L == 0 and C % L == 0
    u_per_w = n_units // NW
    nbb = B // C
    scale = float(math.sqrt(D))
    mesh = plsc.VectorSubcoreMesh(
        core_axis_name="c", subcore_axis_name="s",
        num_cores=NC, num_subcores=NS)

    @functools.partial(
        pl.kernel,
        out_type=jax.ShapeDtypeStruct((T, D, B), jnp.float32),
        mesh=mesh,
        scratch_types=[
            [pltpu.VMEM((C,), jnp.int32) for _ in range(2)],
            [pltpu.VMEM((C, D), jnp.float32) for _ in range(2)],
            [pltpu.VMEM((D, C), jnp.float32) for _ in range(2)],
            [pltpu.SemaphoreType.DMA for _ in range(2)],
            [pltpu.SemaphoreType.DMA for _ in range(2)],
            [pltpu.SemaphoreType.DMA for _ in range(2)],
        ],
        compiler_params=pltpu.CompilerParams(
            use_tc_tiling_on_sc=True, needs_layout_passes=False),
    )
    def emb(tok_hbm, tab_hbm, out_hbm, idx_bufs, row_bufs,
            tr_bufs, isems, gsems, osems):
        wid = lax.axis_index("s") * NC + lax.axis_index("c")
        u_base = wid * u_per_w

        def unit_tb(u):
            return u // nbb, (u % nbb) * C

        def idx_copy(u, b):
            t, b0 = unit_tb(u)
            return pltpu.make_async_copy(
                tok_hbm.at[t, pl.ds(b0, C)], idx_bufs[b], isems[b])

        def start_row_fetches(b):
            def grp(k, carry):
                v16 = idx_bufs[b][pl.ds(k * L, L)]
                for j in range(L):
                    pltpu.make_async_copy(
                        tab_hbm.at[pl.ds(v16[j], 1), :],
                        row_bufs[b].at[pl.ds(k * L + j, 1), :],
                        gsems[b]).start()
                return carry
            lax.fori_loop(0, C // L, grp, 0)

        def drain_row_fetches(b):
            pltpu.make_async_copy(
                tab_hbm.at[pl.ds(0, C), :], row_bufs[b], gsems[b]).wait()

        def out_copy(u, b):
            t, b0 = unit_tb(u)
            return pltpu.make_async_copy(
                tr_bufs[b], out_hbm.at[t, :, pl.ds(b0, C)], osems[b])

        idx_copy(u_base, 0).start()
        idx_copy(u_base + 1, 1).start()
        idx_copy(u_base, 0).wait()
        start_row_fetches(0)

        assert u_per_w % 2 == 0

        def step(i2, carry):
            for b in range(2):
                i = i2 * 2 + b
                u = u_base + i
                nb = 1 - b
                @pl.when(i + 1 < u_per_w)
                def _wait_next_idx():
                    idx_copy(u + 1, nb).wait()
                drain_row_fetches(b)

                @pl.when(i >= 2)
                def _drain_out():
                    out_copy(u - 2, b).wait()

                def col(c, carry):
                    lanes = lax.iota(jnp.int32, L)
                    feat = (c + lanes) & (D - 1)

                    @plsc.parallel_loop(0, C // L, 1, unroll=4)
                    def _grp(k):
                        rows = lanes + (k * L)
                        v = plsc.load_gather(row_bufs[b], [rows, feat])
                        plsc.store_scatter(tr_bufs[b], [feat, rows], v * scale)

                    @pl.when(jnp.logical_and(c < C // L, i + 1 < u_per_w))
                    def _issue_next_rows():
                        v16 = idx_bufs[nb][pl.ds(c * L, L)]
                        for j in range(L):
                            pltpu.make_async_copy(
                                tab_hbm.at[pl.ds(v16[j], 1), :],
                                row_bufs[nb].at[pl.ds(c * L + j, 1), :],
                                gsems[nb]).start()
                    return carry

                lax.fori_loop(0, D, col, 0)
                out_copy(u, b).start()
                @pl.when(i + 2 < u_per_w)
                def _stage_next():
                    idx_copy(u + 2, b).start()
            return carry

        lax.fori_loop(0, u_per_w // 2, step, 0)
        out_copy(u_base + u_per_w - 2, 0).wait()
        out_copy(u_base + u_per_w - 1, 1).wait()

    return emb


@jax.jit
def kernel(tokens, table):
    B, T = tokens.shape
    V, D = table.shape
    tokt = tokens.T.astype(jnp.int32)
    outt = _build(T, B, V, D, 256)(tokt, table)
    return outt.transpose(2, 0, 1)
